# Initial kernel scaffold; baseline (speedup 1.0000x reference)
import jax
import jax.numpy as jnp
from jax.experimental import pallas as pl


def kernel(X, emb_table, lin1_w, lin1_b):
    raise NotImplementedError("write your pallas kernel here")


# speedup vs baseline: 100.0995x; 100.0995x over previous
import functools

import jax
import jax.numpy as jnp
from jax import lax
from jax.experimental import pallas as pl
from jax.experimental.pallas import tpu as pltpu
from jax.experimental.pallas import tpu_sc as plsc

NC = 2
NS = 16
L = 16
NW = NC * NS


def _build(n_total, d_pad, n_chunks, interpret=False):
    per_w = n_total // NW
    chunk = per_w // n_chunks
    assert per_w * NW == n_total and chunk * n_chunks == per_w
    assert chunk % L == 0
    mesh = plsc.VectorSubcoreMesh(core_axis_name="c", subcore_axis_name="s")

    @functools.partial(
        pl.kernel,
        out_type=jax.ShapeDtypeStruct((n_total,), jnp.float32),
        mesh=mesh,
        scratch_types=[
            pltpu.VMEM((d_pad * L,), jnp.float32),
            pltpu.VMEM((d_pad,), jnp.float32),
            pltpu.VMEM((L,), jnp.float32),
            pltpu.VMEM((chunk,), jnp.int32),
            pltpu.VMEM((chunk,), jnp.float32),
        ],
        compiler_params=pltpu.CompilerParams(needs_layout_passes=False),
        interpret=interpret,
    )
    def emb_kernel(x_hbm, emb_hbm, w_hbm, out_hbm, emb_v, w_v, s_ref, idx_v, out_v):
        wid = lax.axis_index("s") * NC + lax.axis_index("c")
        pltpu.sync_copy(emb_hbm, emb_v)
        pltpu.sync_copy(w_hbm, w_v)

        def dot_body(j, acc):
            wv = w_v[pl.ds(j * L, L)]
            for k in range(L):
                e = emb_v[pl.ds(j * L * L + k * L, L)]
                acc = acc + jnp.maximum(e, 0.0) * wv[k]
            return acc

        s_ref[...] = lax.fori_loop(
            0, d_pad // L, dot_body, jnp.zeros((L,), jnp.float32))

        base = wid * per_w
        for c in range(n_chunks):
            off = base + c * chunk
            pltpu.sync_copy(x_hbm.at[pl.ds(off, chunk)], idx_v)

            def body(i, carry):
                idx = idx_v[pl.ds(i * L, L)]
                out_v[pl.ds(i * L, L)] = plsc.load_gather(s_ref, [idx])
                return carry

            lax.fori_loop(0, chunk // L, body, 0)
            pltpu.sync_copy(out_v, out_hbm.at[pl.ds(off, chunk)])

    return emb_kernel


def kernel(X, emb_table, lin1_w, lin1_b):
    b, l = X.shape
    vocab, d = emb_table.shape
    n_total = b * l
    d_pad = ((d + 1 + L - 1) // L) * L
    emb_t = jnp.zeros((d_pad, L), jnp.float32)
    emb_t = emb_t.at[:d, :vocab].set(emb_table.astype(jnp.float32).T)
    emb_t = emb_t.at[d, :].set(1.0)
    w_pad = jnp.zeros((d_pad,), jnp.float32)
    w_pad = w_pad.at[:d].set(lin1_w[0].astype(jnp.float32))
    w_pad = w_pad.at[d].set(lin1_b[0].astype(jnp.float32))

    x_flat = X.reshape(-1).astype(jnp.int32)
    fn = _build(n_total, d_pad, n_chunks=4)
    out = fn(x_flat, emb_t.reshape(-1), w_pad)
    return out.reshape(b, l)
